# baseline (speedup 1.0000x reference)
import jax
import jax.numpy as jnp
from jax import lax
from jax.experimental import pallas as pl
from jax.experimental.pallas import tpu as pltpu
from jax.experimental.pallas import tpu_sc as plsc

_NUM_CLASSES = 90
_C_PAD = 96
_FEA = 128
_BATCH = 16384
_MARGIN = 1.0
_NEG_BIG = -1e30

_NC, _NS = 2, 16
_NW = _NC * _NS
_ROWS_PER_W = _BATCH // _NW
_GROUPS = _ROWS_PER_W // 16

_B_BLK = 2048
_CHUNK = 128


def _tc_maxneg_kernel(x_ref, c_ref, lab_ref, m_ref, invc_ref):
    c = c_ref[...]
    nrm = jnp.sqrt(jnp.sum(c * c, axis=1, keepdims=True))
    invn = 1.0 / (nrm + 1e-12)
    nc = c * invn
    d = lax.dot_general(nc, x_ref[...], (((1,), (1,)), ((), ())),
                        preferred_element_type=jnp.float32)
    row = lax.broadcasted_iota(jnp.int32, d.shape, 0)
    lab = lab_ref[0, 0, :]
    own = row == lab[None, :]
    bad = jnp.logical_or(own, row >= _NUM_CLASSES)
    m_ref[0, 0, :] = jnp.max(jnp.where(bad, _NEG_BIG, d), axis=0)
    invc_ref[0, 0, :] = jnp.sum(jnp.where(own, invn, 0.0), axis=0)


def _sc_pos_kernel(x_hbm, lab_hbm, c_hbm, out_hbm, x_v, lab_v, crow_v,
                   p16_v, sem):
    wid = lax.axis_index("s") * _NC + lax.axis_index("c")
    base = wid * _ROWS_PER_W
    pltpu.sync_copy(lab_hbm.at[pl.ds(base, _ROWS_PER_W)], lab_v)

    for chunk in range(_ROWS_PER_W // _CHUNK):
        coff = chunk * _CHUNK
        pltpu.sync_copy(x_hbm.at[pl.ds(base + coff, _CHUNK)], x_v)
        pltpu.async_copy(c_hbm.at[lab_v.at[pl.ds(coff, _CHUNK)]],
                         crow_v, sem).wait()

        def body(r, carry):
            acc = [jnp.zeros((16,), jnp.float32) for _ in range(4)]
            for k in range(_FEA // 16):
                xv = x_v[r, pl.ds(k * 16, 16)]
                cv = crow_v[r, pl.ds(k * 16, 16)]
                acc[k % 4] = acc[k % 4] + xv * cv
            p16_v[coff + r] = (acc[0] + acc[1]) + (acc[2] + acc[3])
            return carry

        lax.fori_loop(0, _CHUNK, body, 0)

    pltpu.sync_copy(p16_v, out_hbm.at[pl.ds(base, _ROWS_PER_W)])


def _tc_combine_kernel(m_ref, invc_ref, p16_ref, out_ref):
    i = pl.program_id(0)
    m = m_ref[0, 0, :]
    p = jnp.sum(p16_ref[0], axis=1) * invc_ref[0, 0, :]
    blk = jnp.sum(jnp.maximum(_MARGIN + m - p, 0.0)) * (1.0 / _BATCH)

    @pl.when(i == 0)
    def _():
        out_ref[0, 0] = 0.0

    out_ref[0, 0] += blk


def kernel(x, labels, centers):
    labels = labels.astype(jnp.int32)
    cpad = jnp.pad(centers, ((0, _C_PAD - _NUM_CLASSES), (0, 0)))
    lab3 = labels.reshape(_BATCH // _B_BLK, 1, _B_BLK)

    m, invc = pl.pallas_call(
        _tc_maxneg_kernel,
        grid=(_BATCH // _B_BLK,),
        in_specs=[
            pl.BlockSpec((_B_BLK, _FEA), lambda i: (i, 0)),
            pl.BlockSpec((_C_PAD, _FEA), lambda i: (0, 0)),
            pl.BlockSpec((1, 1, _B_BLK), lambda i: (i, 0, 0)),
        ],
        out_specs=[
            pl.BlockSpec((1, 1, _B_BLK), lambda i: (i, 0, 0)),
            pl.BlockSpec((1, 1, _B_BLK), lambda i: (i, 0, 0)),
        ],
        out_shape=[
            jax.ShapeDtypeStruct((_BATCH // _B_BLK, 1, _B_BLK), jnp.float32),
            jax.ShapeDtypeStruct((_BATCH // _B_BLK, 1, _B_BLK), jnp.float32),
        ],
    )(x, cpad, lab3)

    praw16 = pl.kernel(
        _sc_pos_kernel,
        out_type=jax.ShapeDtypeStruct((_BATCH, 16), jnp.float32),
        mesh=plsc.VectorSubcoreMesh(core_axis_name="c", subcore_axis_name="s"),
        scratch_types=[
            pltpu.VMEM((_CHUNK, _FEA), jnp.float32),
            pltpu.VMEM((_ROWS_PER_W,), jnp.int32),
            pltpu.VMEM((_CHUNK, _FEA), jnp.float32),
            pltpu.VMEM((_ROWS_PER_W, 16), jnp.float32),
            pltpu.SemaphoreType.DMA,
        ],
    )(x, labels, cpad)

    p3 = praw16.reshape(_BATCH // _B_BLK, _B_BLK, 16)

    loss = pl.pallas_call(
        _tc_combine_kernel,
        grid=(_BATCH // _B_BLK,),
        in_specs=[
            pl.BlockSpec((1, 1, _B_BLK), lambda i: (i, 0, 0)),
            pl.BlockSpec((1, 1, _B_BLK), lambda i: (i, 0, 0)),
            pl.BlockSpec((1, _B_BLK, 16), lambda i: (i, 0, 0)),
        ],
        out_specs=pl.BlockSpec(memory_space=pltpu.SMEM),
        out_shape=jax.ShapeDtypeStruct((1, 1), jnp.float32),
    )(m, invc, p3)

    return loss[0, 0]


# speedup vs baseline: 1.3927x; 1.3927x over previous
import jax
import jax.numpy as jnp
from jax import lax
from jax.experimental import pallas as pl
from jax.experimental.pallas import tpu as pltpu
from jax.experimental.pallas import tpu_sc as plsc

_NUM_CLASSES = 90
_C_PAD = 96
_FEA = 128
_BATCH = 16384
_MARGIN = 1.0
_NEG_BIG = -1e30

_NC, _NS = 2, 16
_NW = _NC * _NS
_B_BLK = 2048
_N_BLK = _BATCH // _B_BLK
_SC_ROWS = 4096
_SC_BASE = _BATCH - _SC_ROWS
_SC_BLK0 = _SC_BASE // _B_BLK
_RPW = _SC_ROWS // _NW


def _tc_maxneg_kernel(x_ref, c_ref, lab_ref, m_ref, ptc_ref, invc_ref):
    c = c_ref[...]
    nrm = jnp.sqrt(jnp.sum(c * c, axis=1, keepdims=True))
    invn = 1.0 / (nrm + 1e-12)
    nc = c * invn
    d = lax.dot_general(nc, x_ref[...], (((1,), (1,)), ((), ())),
                        preferred_element_type=jnp.float32)
    row = lax.broadcasted_iota(jnp.int32, d.shape, 0)
    lab = lab_ref[0, 0, :]
    own = row == lab[None, :]
    bad = jnp.logical_or(own, row >= _NUM_CLASSES)
    m_ref[0, 0, :] = jnp.max(jnp.where(bad, _NEG_BIG, d), axis=0)
    ptc_ref[0, 0, :] = jnp.sum(jnp.where(own, d, 0.0), axis=0)
    invc_ref[0, 0, :] = jnp.sum(jnp.where(own, invn, 0.0), axis=0)


def _sc_pos_kernel(x_hbm, lab_hbm, c_hbm, out_hbm, x_v, lab_v, crow_v,
                   p_v, sem):
    wid = lax.axis_index("s") * _NC + lax.axis_index("c")
    base = _SC_BASE + wid * _RPW
    pltpu.sync_copy(x_hbm.at[pl.ds(base, _RPW)], x_v)
    pltpu.sync_copy(lab_hbm.at[pl.ds(base, _RPW)], lab_v)
    pltpu.async_copy(c_hbm.at[lab_v], crow_v, sem).wait()

    def body(r, carry):
        acc = [jnp.zeros((16,), jnp.float32) for _ in range(4)]
        for k in range(_FEA // 16):
            xv = x_v[r, pl.ds(k * 16, 16)]
            cv = crow_v[r, pl.ds(k * 16, 16)]
            acc[k % 4] = acc[k % 4] + xv * cv
        p_v[r] = (acc[0] + acc[1]) + (acc[2] + acc[3])
        return carry

    lax.fori_loop(0, _RPW, body, 0)
    pltpu.sync_copy(p_v, out_hbm.at[pl.ds(wid * _RPW, _RPW)])


def _tc_combine_kernel(m_ref, ptc_ref, invc_ref, psc_ref, out_ref):
    i = pl.program_id(0)
    m = m_ref[0, 0, :]
    p_tc = ptc_ref[0, 0, :]
    p_sc = jnp.sum(psc_ref[0], axis=1) * invc_ref[0, 0, :]
    p = jnp.where(i < _SC_BLK0, p_tc, p_sc)
    blk = jnp.sum(jnp.maximum(_MARGIN + m - p, 0.0)) * (1.0 / _BATCH)

    @pl.when(i == 0)
    def _():
        out_ref[0, 0] = 0.0

    out_ref[0, 0] += blk


def kernel(x, labels, centers):
    labels = labels.astype(jnp.int32)
    cpad = jnp.pad(centers, ((0, _C_PAD - _NUM_CLASSES), (0, 0)))
    lab3 = labels.reshape(_N_BLK, 1, _B_BLK)

    m, ptc, invc = pl.pallas_call(
        _tc_maxneg_kernel,
        grid=(_N_BLK,),
        in_specs=[
            pl.BlockSpec((_B_BLK, _FEA), lambda i: (i, 0)),
            pl.BlockSpec((_C_PAD, _FEA), lambda i: (0, 0)),
            pl.BlockSpec((1, 1, _B_BLK), lambda i: (i, 0, 0)),
        ],
        out_specs=[
            pl.BlockSpec((1, 1, _B_BLK), lambda i: (i, 0, 0)),
            pl.BlockSpec((1, 1, _B_BLK), lambda i: (i, 0, 0)),
            pl.BlockSpec((1, 1, _B_BLK), lambda i: (i, 0, 0)),
        ],
        out_shape=[
            jax.ShapeDtypeStruct((_N_BLK, 1, _B_BLK), jnp.float32),
            jax.ShapeDtypeStruct((_N_BLK, 1, _B_BLK), jnp.float32),
            jax.ShapeDtypeStruct((_N_BLK, 1, _B_BLK), jnp.float32),
        ],
    )(x, cpad, lab3)

    psc = pl.kernel(
        _sc_pos_kernel,
        out_type=jax.ShapeDtypeStruct((_SC_ROWS, 16), jnp.float32),
        mesh=plsc.VectorSubcoreMesh(core_axis_name="c", subcore_axis_name="s"),
        scratch_types=[
            pltpu.VMEM((_RPW, _FEA), jnp.float32),
            pltpu.VMEM((_RPW,), jnp.int32),
            pltpu.VMEM((_RPW, _FEA), jnp.float32),
            pltpu.VMEM((_RPW, 16), jnp.float32),
            pltpu.SemaphoreType.DMA,
        ],
    )(x, labels, cpad)

    psc3 = psc.reshape(_SC_ROWS // _B_BLK, _B_BLK, 16)

    loss = pl.pallas_call(
        _tc_combine_kernel,
        grid=(_N_BLK,),
        in_specs=[
            pl.BlockSpec((1, 1, _B_BLK), lambda i: (i, 0, 0)),
            pl.BlockSpec((1, 1, _B_BLK), lambda i: (i, 0, 0)),
            pl.BlockSpec((1, 1, _B_BLK), lambda i: (i, 0, 0)),
            pl.BlockSpec((1, _B_BLK, 16),
                         lambda i: (jnp.maximum(i - _SC_BLK0, 0), 0, 0)),
        ],
        out_specs=pl.BlockSpec(memory_space=pltpu.SMEM),
        out_shape=jax.ShapeDtypeStruct((1, 1), jnp.float32),
    )(m, ptc, invc, psc3)

    return loss[0, 0]


# speedup vs baseline: 1.5681x; 1.1259x over previous
import jax
import jax.numpy as jnp
from jax import lax
from jax.experimental import pallas as pl
from jax.experimental.pallas import tpu as pltpu
from jax.experimental.pallas import tpu_sc as plsc

_NUM_CLASSES = 90
_C_PAD = 96
_FEA = 128
_BATCH = 16384
_MARGIN = 1.0
_NEG_BIG = -1e30

_NC, _NS = 2, 16
_NW = _NC * _NS
_N_BLK = 8
_HB = _BATCH // (2 * _N_BLK)
_SC_ROWS = 4096
_SC_BASE = _BATCH - _SC_ROWS
_SC_HBLK0 = (_SC_BASE - _BATCH // 2) // _HB
_RPW = _SC_ROWS // _NW


def _tc_maxneg_kernel(xa_ref, xb_ref, c_ref, laba_ref, labb_ref,
                      ma_ref, mb_ref, pa_ref, pb_ref, ib_ref):
    c = c_ref[...]
    nrm = jnp.sqrt(jnp.sum(c * c, axis=1, keepdims=True))
    invn = 1.0 / (nrm + 1e-12)
    nc = c * invn

    def half(x_ref, lab_ref, m_ref, p_ref, i_ref):
        d = lax.dot_general(nc, x_ref[...], (((1,), (1,)), ((), ())),
                            preferred_element_type=jnp.float32)
        row = lax.broadcasted_iota(jnp.int32, d.shape, 0)
        lab = lab_ref[0, 0, :]
        own = row == lab[None, :]
        bad = jnp.logical_or(own, row >= _NUM_CLASSES)
        m_ref[0, 0, :] = jnp.max(jnp.where(bad, _NEG_BIG, d), axis=0)
        p_ref[0, 0, :] = jnp.sum(jnp.where(own, d, 0.0), axis=0)
        if i_ref is not None:
            i_ref[0, 0, :] = jnp.sum(jnp.where(own, invn, 0.0), axis=0)

    half(xa_ref, laba_ref, ma_ref, pa_ref, None)
    half(xb_ref, labb_ref, mb_ref, pb_ref, ib_ref)


def _sc_pos_kernel(x_hbm, lab_hbm, c_hbm, out_hbm, x_v, lab_v, crow_v,
                   p_v, sem):
    wid = lax.axis_index("s") * _NC + lax.axis_index("c")
    base = _SC_BASE + wid * _RPW
    pltpu.sync_copy(x_hbm.at[pl.ds(base, _RPW)], x_v)
    pltpu.sync_copy(lab_hbm.at[pl.ds(base, _RPW)], lab_v)
    pltpu.async_copy(c_hbm.at[lab_v], crow_v, sem).wait()

    def body(r, carry):
        acc = [jnp.zeros((16,), jnp.float32) for _ in range(4)]
        for k in range(_FEA // 16):
            xv = x_v[r, pl.ds(k * 16, 16)]
            cv = crow_v[r, pl.ds(k * 16, 16)]
            acc[k % 4] = acc[k % 4] + xv * cv
        p_v[r] = (acc[0] + acc[1]) + (acc[2] + acc[3])
        return carry

    lax.fori_loop(0, _RPW, body, 0)
    pltpu.sync_copy(p_v, out_hbm.at[pl.ds(wid * _RPW, _RPW)])


def _tc_combine_kernel(ma_ref, pa_ref, mb_ref, pb_ref, ib_ref, psc_ref,
                       out_ref):
    i = pl.program_id(0)
    lo = jnp.maximum(_MARGIN + ma_ref[0, 0, :] - pa_ref[0, 0, :], 0.0)
    p_sc = jnp.sum(psc_ref[0], axis=1) * ib_ref[0, 0, :]
    p_hi = jnp.where(i < _SC_HBLK0, pb_ref[0, 0, :], p_sc)
    hi = jnp.maximum(_MARGIN + mb_ref[0, 0, :] - p_hi, 0.0)
    blk = (jnp.sum(lo) + jnp.sum(hi)) * (1.0 / _BATCH)

    @pl.when(i == 0)
    def _():
        out_ref[0, 0] = 0.0

    out_ref[0, 0] += blk


def kernel(x, labels, centers):
    labels = labels.astype(jnp.int32)
    cpad = jnp.pad(centers, ((0, _C_PAD - _NUM_CLASSES), (0, 0)))
    lab3 = labels.reshape(_BATCH // _HB, 1, _HB)

    lo_blk = pl.BlockSpec((1, 1, _HB), lambda i: (i, 0, 0))
    hi_blk = pl.BlockSpec((1, 1, _HB), lambda i: (i + _N_BLK, 0, 0))
    half_sd = jax.ShapeDtypeStruct((_N_BLK, 1, _HB), jnp.float32)

    ma, mb, pa, pb, ib = pl.pallas_call(
        _tc_maxneg_kernel,
        grid=(_N_BLK,),
        in_specs=[
            pl.BlockSpec((_HB, _FEA), lambda i: (i, 0)),
            pl.BlockSpec((_HB, _FEA), lambda i: (i + _N_BLK, 0)),
            pl.BlockSpec((_C_PAD, _FEA), lambda i: (0, 0)),
            lo_blk,
            hi_blk,
        ],
        out_specs=[lo_blk, lo_blk, lo_blk, lo_blk, lo_blk],
        out_shape=[half_sd] * 5,
    )(x, x, cpad, lab3, lab3)

    psc = pl.kernel(
        _sc_pos_kernel,
        out_type=jax.ShapeDtypeStruct((_SC_ROWS, 16), jnp.float32),
        mesh=plsc.VectorSubcoreMesh(core_axis_name="c", subcore_axis_name="s"),
        scratch_types=[
            pltpu.VMEM((_RPW, _FEA), jnp.float32),
            pltpu.VMEM((_RPW,), jnp.int32),
            pltpu.VMEM((_RPW, _FEA), jnp.float32),
            pltpu.VMEM((_RPW, 16), jnp.float32),
            pltpu.SemaphoreType.DMA,
        ],
    )(x, labels, cpad)

    psc3 = psc.reshape(_SC_ROWS // _HB, _HB, 16)

    loss = pl.pallas_call(
        _tc_combine_kernel,
        grid=(_N_BLK,),
        in_specs=[
            lo_blk,
            lo_blk,
            lo_blk,
            lo_blk,
            lo_blk,
            pl.BlockSpec((1, _HB, 16),
                         lambda i: (jnp.maximum(i - _SC_HBLK0, 0), 0, 0)),
        ],
        out_specs=pl.BlockSpec(memory_space=pltpu.SMEM),
        out_shape=jax.ShapeDtypeStruct((1, 1), jnp.float32),
    )(ma, pa, mb, pb, ib, psc3)

    return loss[0, 0]


# speedup vs baseline: 1.9065x; 1.2158x over previous
import jax
import jax.numpy as jnp
from jax import lax
from jax.experimental import pallas as pl
from jax.experimental.pallas import tpu as pltpu
from jax.experimental.pallas import tpu_sc as plsc

_NUM_CLASSES = 90
_C_PAD = 96
_FEA = 128
_BATCH = 16384
_MARGIN = 1.0
_NEG_BIG = -1e30

_NC, _NS = 2, 16
_NW = _NC * _NS
_N_BLK = 4
_HB = _BATCH // (2 * _N_BLK)
_SC_ROWS = 4096
_SC_BASE = _BATCH - _SC_ROWS
_SC_HBLK0 = (_SC_BASE - _BATCH // 2) // _HB
_RPW = _SC_ROWS // _NW


def _tc_main_kernel(xa_ref, xb_ref, c_ref, laba_ref, labb_ref,
                    part_ref, msc_ref, isc_ref):
    i = pl.program_id(0)
    c = c_ref[...]
    nrm = jnp.sqrt(jnp.sum(c * c, axis=1, keepdims=True))
    invn = 1.0 / (nrm + 1e-12)
    nc = c * invn

    def half(x_ref, lab_ref):
        d = lax.dot_general(nc, x_ref[...], (((1,), (1,)), ((), ())),
                            preferred_element_type=jnp.float32)
        row = lax.broadcasted_iota(jnp.int32, d.shape, 0)
        lab = lab_ref[0, 0, :]
        own = row == lab[None, :]
        bad = jnp.logical_or(own, row >= _NUM_CLASSES)
        m = jnp.max(jnp.where(bad, _NEG_BIG, d), axis=0)
        p = jnp.sum(jnp.where(own, d, 0.0), axis=0)
        return d, own, m, p

    _, _, m_lo, p_lo = half(xa_ref, laba_ref)
    _, own_hi, m_hi, p_hi = half(xb_ref, labb_ref)

    part_lo = jnp.sum(jnp.maximum(_MARGIN + m_lo - p_lo, 0.0))
    part_hi = jnp.sum(jnp.maximum(_MARGIN + m_hi - p_hi, 0.0))
    part = part_lo + jnp.where(i < _SC_HBLK0, part_hi, 0.0)

    msc_ref[0, 0, :] = m_hi
    isc_ref[0, 0, :] = jnp.sum(jnp.where(own_hi, invn, 0.0), axis=0)

    @pl.when(i == 0)
    def _():
        part_ref[0, 0] = 0.0

    part_ref[0, 0] += part


def _sc_pos_kernel(x_hbm, lab_hbm, c_hbm, out_hbm, x_v, lab_v, crow_v,
                   p_v, sem):
    wid = lax.axis_index("s") * _NC + lax.axis_index("c")
    base = _SC_BASE + wid * _RPW
    pltpu.sync_copy(x_hbm.at[pl.ds(base, _RPW)], x_v)
    pltpu.sync_copy(lab_hbm.at[pl.ds(base, _RPW)], lab_v)
    pltpu.async_copy(c_hbm.at[lab_v], crow_v, sem).wait()

    def body(r, carry):
        acc = [jnp.zeros((16,), jnp.float32) for _ in range(4)]
        for k in range(_FEA // 16):
            xv = x_v[r, pl.ds(k * 16, 16)]
            cv = crow_v[r, pl.ds(k * 16, 16)]
            acc[k % 4] = acc[k % 4] + xv * cv
        p_v[r] = (acc[0] + acc[1]) + (acc[2] + acc[3])
        return carry

    lax.fori_loop(0, _RPW, body, 0)
    pltpu.sync_copy(p_v, out_hbm.at[pl.ds(wid * _RPW, _RPW)])


def _tc_combine_kernel(part_ref, msc_ref, isc_ref, psc_ref, out_ref):
    i = pl.program_id(0)
    ones = jnp.ones((1, 16), jnp.float32)
    p16 = psc_ref[0]
    praw = lax.dot_general(ones, p16, (((1,), (1,)), ((), ())),
                           preferred_element_type=jnp.float32)[0]
    p = praw * isc_ref[0, 0, :]
    blk = jnp.sum(jnp.maximum(_MARGIN + msc_ref[0, 0, :] - p, 0.0))

    @pl.when(i == 0)
    def _():
        out_ref[0, 0] = part_ref[0, 0] * (1.0 / _BATCH)

    out_ref[0, 0] += blk * (1.0 / _BATCH)


def kernel(x, labels, centers):
    labels = labels.astype(jnp.int32)
    cpad = jnp.pad(centers, ((0, _C_PAD - _NUM_CLASSES), (0, 0)))
    lab3 = labels.reshape(_BATCH // _HB, 1, _HB)

    lo_blk = pl.BlockSpec((1, 1, _HB), lambda i: (i, 0, 0))
    hi_blk = pl.BlockSpec((1, 1, _HB), lambda i: (i + _N_BLK, 0, 0))
    sc_blk = pl.BlockSpec((1, 1, _HB),
                          lambda i: (jnp.maximum(i - _SC_HBLK0, 0), 0, 0))
    sc_sd = jax.ShapeDtypeStruct((_SC_ROWS // _HB, 1, _HB), jnp.float32)

    part, msc, isc = pl.pallas_call(
        _tc_main_kernel,
        grid=(_N_BLK,),
        in_specs=[
            pl.BlockSpec((_HB, _FEA), lambda i: (i, 0)),
            pl.BlockSpec((_HB, _FEA), lambda i: (i + _N_BLK, 0)),
            pl.BlockSpec((_C_PAD, _FEA), lambda i: (0, 0)),
            lo_blk,
            hi_blk,
        ],
        out_specs=[
            pl.BlockSpec(memory_space=pltpu.SMEM),
            sc_blk,
            sc_blk,
        ],
        out_shape=[
            jax.ShapeDtypeStruct((1, 1), jnp.float32),
            sc_sd,
            sc_sd,
        ],
    )(x, x, cpad, lab3, lab3)

    psc = pl.kernel(
        _sc_pos_kernel,
        out_type=jax.ShapeDtypeStruct((_SC_ROWS, 16), jnp.float32),
        mesh=plsc.VectorSubcoreMesh(core_axis_name="c", subcore_axis_name="s"),
        scratch_types=[
            pltpu.VMEM((_RPW, _FEA), jnp.float32),
            pltpu.VMEM((_RPW,), jnp.int32),
            pltpu.VMEM((_RPW, _FEA), jnp.float32),
            pltpu.VMEM((_RPW, 16), jnp.float32),
            pltpu.SemaphoreType.DMA,
        ],
    )(x, labels, cpad)

    psc3 = psc.reshape(_SC_ROWS // _HB, _HB, 16)

    loss = pl.pallas_call(
        _tc_combine_kernel,
        grid=(_SC_ROWS // _HB,),
        in_specs=[
            pl.BlockSpec(memory_space=pltpu.SMEM),
            pl.BlockSpec((1, 1, _HB), lambda i: (i, 0, 0)),
            pl.BlockSpec((1, 1, _HB), lambda i: (i, 0, 0)),
            pl.BlockSpec((1, _HB, 16), lambda i: (i, 0, 0)),
        ],
        out_specs=pl.BlockSpec(memory_space=pltpu.SMEM),
        out_shape=jax.ShapeDtypeStruct((1, 1), jnp.float32),
    )(part, msc, isc, psc3)

    return loss[0, 0]
